# baseline (speedup 1.0000x reference)
import jax
import jax.numpy as jnp
from jax.experimental import pallas as pl

N_CB = 4
D_IN = 512
CB_SIZE = 1024
CB_DIM = 8

_HI = jax.lax.Precision.HIGHEST


def _rvq_kernel(z_ref, win_ref, bin_ref, cb_ref, wout_ref, bout_ref,
                zq_ref, codes_ref, loss_ref):
    b = pl.program_id(0)
    res = z_ref[0]
    T = res.shape[1]
    zq_acc = jnp.zeros_like(res)
    loss_acc = jnp.zeros((), jnp.float32)

    for i in range(N_CB):
        w_in = win_ref[i]
        cb = cb_ref[i]
        w_out = wout_ref[i]
        b_out = bout_ref[i]

        ze_cm = jax.lax.dot_general(w_in, res, (((1,), (0,)), ((), ())))
        ze_cm = ze_cm + bin_ref[0, i][:, None]
        ze = ze_cm.T

        ze_n = ze / jnp.clip(jnp.sqrt(jnp.sum(ze * ze, axis=1, keepdims=True)),
                             1e-12, None)
        cb_n = cb / jnp.clip(jnp.sqrt(jnp.sum(cb * cb, axis=1, keepdims=True)),
                             1e-12, None)
        m = jax.lax.dot_general(ze_n, cb_n, (((1,), (1,)), ((), ())))
        s_e = jnp.sum(ze_n * ze_n, axis=1, keepdims=True)
        s_c = jnp.sum(cb_n * cb_n, axis=1)[None, :]
        dist = s_e - 2.0 * m + s_c
        idx = jnp.argmax(-dist, axis=1)

        onehot = (jax.lax.broadcasted_iota(jnp.int32, (T, CB_SIZE), 1)
                  == idx[:, None]).astype(jnp.float32)
        zq_small = jax.lax.dot_general(onehot, cb, (((1,), (0,)), ((), ())),
                                       precision=_HI)

        diff = ze - zq_small
        loss_acc = loss_acc + jnp.sum(diff * diff)

        zq_out = jax.lax.dot_general(w_out, zq_small.T, (((1,), (0,)), ((), ())))
        zq_out = zq_out + b_out[:, None]

        zq_acc = zq_acc + zq_out
        res = res - zq_out
        codes_ref[0, pl.ds(i, 1), :] = idx.reshape(1, T)

    zq_ref[0] = zq_acc
    scale = 1.0 / (CB_DIM * T)
    val = (loss_acc * scale).reshape(1, 1)
    @pl.when(b == 0)
    def _init():
        loss_ref[...] = val
    @pl.when(b != 0)
    def _acc():
        loss_ref[...] = loss_ref[...] + val


@jax.jit
def kernel(z, W_in, b_in, codebooks, W_out, b_out):
    B, Din, T = z.shape
    zq, codes, loss = pl.pallas_call(
        _rvq_kernel,
        grid=(B,),
        in_specs=[
            pl.BlockSpec((1, Din, T), lambda b: (b, 0, 0)),
            pl.BlockSpec((N_CB, CB_DIM, Din), lambda b: (0, 0, 0)),
            pl.BlockSpec((1, N_CB, CB_DIM), lambda b: (0, 0, 0)),
            pl.BlockSpec((N_CB, CB_SIZE, CB_DIM), lambda b: (0, 0, 0)),
            pl.BlockSpec((N_CB, Din, CB_DIM), lambda b: (0, 0, 0)),
            pl.BlockSpec((N_CB, Din), lambda b: (0, 0)),
        ],
        out_specs=[
            pl.BlockSpec((1, Din, T), lambda b: (b, 0, 0)),
            pl.BlockSpec((1, N_CB, T), lambda b: (b, 0, 0)),
            pl.BlockSpec((1, 1), lambda b: (0, 0)),
        ],
        out_shape=[
            jax.ShapeDtypeStruct((B, Din, T), jnp.float32),
            jax.ShapeDtypeStruct((B, N_CB, T), jnp.int32),
            jax.ShapeDtypeStruct((1, 1), jnp.float32),
        ],
    )(z, W_in, b_in[None], codebooks, W_out, b_out)
    loss_scalar = (loss[0, 0] / B).astype(z.dtype)
    return zq, codes, loss_scalar, loss_scalar
